# baseline (speedup 1.0000x reference)
import functools

import jax
import jax.numpy as jnp
from jax.experimental import pallas as pl
from jax.experimental.pallas import tpu as pltpu

K = 16
BN = 256


def _gaencode_block(xyz_rows_ref, xyz_all_ref, feats_ref, wt_ref, out_ref):
    rows = xyz_rows_ref[0]
    allp = xyz_all_ref[0]
    xx_rows = jnp.sum(rows * rows, axis=1, keepdims=True)
    xx_all = jnp.sum(allp * allp, axis=0, keepdims=True)
    mm = jax.lax.dot_general(
        rows.astype(jnp.bfloat16), allp.astype(jnp.bfloat16),
        (((1,), (0,)), ((), ())), preferred_element_type=jnp.float32)
    inner = -2.0 * mm
    acc = -xx_rows - inner - xx_all
    neg_inf = jnp.float32(-jnp.inf)
    t = jnp.full((rows.shape[0], 1), jnp.inf, jnp.float32)

    def body(_, t):
        masked = jnp.where(acc < t, acc, neg_inf)
        return jnp.max(masked, axis=1, keepdims=True)

    t = jax.lax.fori_loop(0, K, body, t)
    maskf = (acc >= t).astype(jnp.float32)
    cnt = jnp.sum(maskf, axis=1, keepdims=True)
    pooled = jnp.dot(maskf.astype(jnp.bfloat16),
                     feats_ref[0].astype(jnp.bfloat16),
                     preferred_element_type=jnp.float32)
    pooled = pooled * (1.0 / cnt)
    out_ref[0] = jnp.dot(pooled.astype(jnp.bfloat16),
                         wt_ref[...].astype(jnp.bfloat16),
                         preferred_element_type=jnp.float32)


@jax.jit
def kernel(xyz_B3N, feats_BNC, W):
    B, _, N = xyz_B3N.shape
    C = feats_BNC.shape[-1]
    xyzT = jnp.transpose(xyz_B3N, (0, 2, 1))
    Wt = jnp.transpose(W)
    grid = (B, N // BN)
    return pl.pallas_call(
        _gaencode_block,
        grid=grid,
        in_specs=[
            pl.BlockSpec((1, BN, 3), lambda b, r: (b, r, 0)),
            pl.BlockSpec((1, 3, N), lambda b, r: (b, 0, 0)),
            pl.BlockSpec((1, N, C), lambda b, r: (b, 0, 0)),
            pl.BlockSpec((C, C), lambda b, r: (0, 0)),
        ],
        out_specs=pl.BlockSpec((1, BN, C), lambda b, r: (b, r, 0)),
        out_shape=jax.ShapeDtypeStruct((B, N, C), jnp.float32),
        compiler_params=pltpu.CompilerParams(
            dimension_semantics=("parallel", "parallel")),
    )(xyzT, xyz_B3N, feats_BNC, Wt)


# speedup vs baseline: 1.4414x; 1.4414x over previous
import functools

import jax
import jax.numpy as jnp
from jax.experimental import pallas as pl
from jax.experimental.pallas import tpu as pltpu

K = 16
BN = 256


def _gaencode_block(xyz_rows_ref, xyz_all_ref, feats_ref, wt_ref, out_ref):
    rows = xyz_rows_ref[0]
    allp = xyz_all_ref[0]
    xx_rows = jnp.sum(rows * rows, axis=1, keepdims=True)
    xx_all = jnp.sum(allp * allp, axis=0, keepdims=True)
    mm = jax.lax.dot_general(
        rows.astype(jnp.bfloat16), allp.astype(jnp.bfloat16),
        (((1,), (0,)), ((), ())), preferred_element_type=jnp.float32)
    inner = -2.0 * mm
    acc = -xx_rows - inner - xx_all
    neg_inf = jnp.float32(-jnp.inf)
    BNr = rows.shape[0]
    NS = acc.shape[1] // 128
    m1 = jnp.full((BNr, 128), -jnp.inf, jnp.float32)
    m2 = jnp.full((BNr, 128), -jnp.inf, jnp.float32)
    for v in range(NS):
        s = acc[:, v * 128:(v + 1) * 128]
        new_m1 = jnp.maximum(m1, s)
        m2 = jnp.maximum(m2, jnp.minimum(m1, s))
        m1 = new_m1

    t = jnp.full((BNr, 1), jnp.inf, jnp.float32)

    def body(_, t):
        a = jnp.where(m1 < t, m1, neg_inf)
        b = jnp.where(m2 < t, m2, neg_inf)
        return jnp.max(jnp.maximum(a, b), axis=1, keepdims=True)

    L = jax.lax.fori_loop(0, K, body, t)
    cnt0 = jnp.sum((acc >= L).astype(jnp.int32), axis=1, keepdims=True)
    rem0 = cnt0 - K

    def asc_cond(carry):
        _, rem = carry
        return jnp.any(rem > 0)

    def asc_body(carry):
        T, rem = carry
        cand = jnp.where(acc >= T, acc, jnp.inf)
        m = jnp.min(cand, axis=1, keepdims=True)
        bump = jax.lax.bitcast_convert_type(
            jax.lax.bitcast_convert_type(m, jnp.int32) - 1, jnp.float32)
        active = rem > 0
        T = jnp.where(active, bump, T)
        rem = jnp.where(active, rem - 1, rem)
        return T, rem

    t, _ = jax.lax.while_loop(asc_cond, asc_body, (L, rem0))
    maskf = (acc >= t).astype(jnp.float32)
    cnt = jnp.sum(maskf, axis=1, keepdims=True)
    pooled = jnp.dot(maskf.astype(jnp.bfloat16),
                     feats_ref[0].astype(jnp.bfloat16),
                     preferred_element_type=jnp.float32)
    pooled = pooled * (1.0 / cnt)
    out_ref[0] = jnp.dot(pooled.astype(jnp.bfloat16),
                         wt_ref[...].astype(jnp.bfloat16),
                         preferred_element_type=jnp.float32)


@jax.jit
def kernel(xyz_B3N, feats_BNC, W):
    B, _, N = xyz_B3N.shape
    C = feats_BNC.shape[-1]
    xyzT = jnp.transpose(xyz_B3N, (0, 2, 1))
    Wt = jnp.transpose(W)
    grid = (B, N // BN)
    return pl.pallas_call(
        _gaencode_block,
        grid=grid,
        in_specs=[
            pl.BlockSpec((1, BN, 3), lambda b, r: (b, r, 0)),
            pl.BlockSpec((1, 3, N), lambda b, r: (b, 0, 0)),
            pl.BlockSpec((1, N, C), lambda b, r: (b, 0, 0)),
            pl.BlockSpec((C, C), lambda b, r: (0, 0)),
        ],
        out_specs=pl.BlockSpec((1, BN, C), lambda b, r: (b, r, 0)),
        out_shape=jax.ShapeDtypeStruct((B, N, C), jnp.float32),
        compiler_params=pltpu.CompilerParams(
            dimension_semantics=("parallel", "parallel")),
    )(xyzT, xyz_B3N, feats_BNC, Wt)
